# baseline (speedup 1.0000x reference)
import jax
import jax.numpy as jnp
from jax.experimental import pallas as pl
from jax.experimental.pallas import tpu as pltpu


def _block_body(A, T, x_ref, wf_ref, wb_ref, bias_ref, acts_ref,
                fwd_ref, bwd_ref, wt_s, bias_s):
    t = pl.program_id(1)

    @pl.when(t == 0)
    def _init():
        wt_s[:A, :] = wf_ref[:, :]
        wt_s[A:, :] = wb_ref[:, :]
        bias_s[:, :] = bias_ref[:, :].T

    x = x_ref[0]
    logits = jnp.dot(wt_s[:, :], x, preferred_element_type=jnp.float32)
    logits = logits + bias_s[:, :]
    Bb = logits.shape[1]
    ids = jax.lax.broadcasted_iota(jnp.int32, (A, Bb), 0)

    def select_prob(l, act):
        m = jnp.max(l, axis=0, keepdims=True)
        e = jnp.exp(l - m)
        s = jnp.sum(e, axis=0, keepdims=True)
        sel = jnp.sum(jnp.where(ids == act, e, 0.0), axis=0, keepdims=True)
        return sel / s

    af = acts_ref[pl.ds(t, 1), :]
    ab = acts_ref[pl.ds(jax.lax.rem(t + T - 1, T), 1), :]
    fwd_ref[pl.ds(t, 1), :] = select_prob(logits[:A, :], af)
    bwd = select_prob(logits[A:, :], ab)
    bwd = jnp.where(ab == 2, 1.0, bwd)

    @pl.when(t > 0)
    def _store_bwd():
        bwd_ref[pl.ds(t - 1, 1), :] = bwd


def kernel(traj, actions, Wf, bf, Wb, bb, answer):
    B, T, D = traj.shape
    A = Wf.shape[1]

    xt = traj.transpose(1, 2, 0)
    wf_t = Wf.T
    wb_t = Wb.T
    bias = jnp.concatenate([bf, bb]).reshape(1, 2 * A)
    actsT = actions.T.astype(jnp.int32)

    J = 2
    Bb = B // J
    fwd, bwd = pl.pallas_call(
        lambda *refs: _block_body(A, T, *refs),
        grid=(J, T),
        in_specs=[
            pl.BlockSpec((1, D, Bb), lambda j, t: (t, 0, j)),
            pl.BlockSpec((A, D), lambda j, t: (0, 0)),
            pl.BlockSpec((A, D), lambda j, t: (0, 0)),
            pl.BlockSpec((1, 2 * A), lambda j, t: (0, 0)),
            pl.BlockSpec((T, Bb), lambda j, t: (0, j)),
        ],
        out_specs=[
            pl.BlockSpec((T, Bb), lambda j, t: (0, j)),
            pl.BlockSpec((T - 1, Bb), lambda j, t: (0, j)),
        ],
        out_shape=[
            jax.ShapeDtypeStruct((T, B), jnp.float32),
            jax.ShapeDtypeStruct((T - 1, B), jnp.float32),
        ],
        scratch_shapes=[
            pltpu.VMEM((2 * A, D), jnp.float32),
            pltpu.VMEM((2 * A, 1), jnp.float32),
        ],
        compiler_params=pltpu.CompilerParams(
            dimension_semantics=("parallel", "arbitrary"),
        ),
    )(xt, wf_t, wb_t, bias, actsT)

    fwd_sel = fwd.T
    back_sel = bwd.T
    rewards = jnp.ones((), jnp.float32)
    return fwd_sel, back_sel, rewards
